# baseline (speedup 1.0000x reference)
import functools
import math

import jax
import jax.numpy as jnp
from jax.experimental import pallas as pl


def _gnn_body(x_ref, w1_ref, b1_ref, w2_ref, b2_ref, o_ref, *, n, alpha, beta, c_b, c_r):
    s, f, _ = x_ref.shape
    h = w1_ref.shape[1]
    xs = x_ref[...]
    w1 = w1_ref[...]
    nc = 200
    xw0 = jax.lax.dot_general(
        xs[:, :, :nc], w1,
        (((1,), (0,)), ((), ())),
        preferred_element_type=jnp.float32,
    )
    xb = xw0[:, 0, :]
    s1 = jnp.dot(jnp.sum(xs, axis=2), w1,
                 preferred_element_type=jnp.float32) - xb
    b1 = b1_ref[...]
    c = b1[None, :] + (2.0 * alpha) * xb
    t0 = jnp.maximum(c[:, None, :] + xw0, 0.0)
    S1 = jnp.sum(t0, axis=1) - t0[:, 0, :]
    for k in range(nc, n, nc):
        xwk = jax.lax.dot_general(
            xs[:, :, k:k + nc], w1,
            (((1,), (0,)), ((), ())),
            preferred_element_type=jnp.float32,
        )
        S1 = S1 + jnp.sum(jnp.maximum(c[:, None, :] + xwk, 0.0), axis=1)
    hb = jnp.maximum(b1[None, :] + (2.0 * alpha) * s1 + (2.0 * beta) * xb, 0.0)
    v = c_b * hb + c_r * S1
    i = pl.program_id(0)
    o_ref[pl.ds(i * s, s), :] = b2_ref[...][None, :] + jnp.dot(
        v, w2_ref[...], preferred_element_type=jnp.float32)


def kernel(x, W1, b1, W2, b2):
    B, N, F = x.shape
    K1, H = W1.shape
    O = W2.shape[1]
    W1h = 0.5 * W1

    alpha = 1.0 / math.sqrt(2.0 * N)
    beta = 1.0 / N
    c_b = ((N - 1) * alpha + beta) / N
    c_r = (alpha + 0.5) / N

    S = 50
    assert B % S == 0
    x_t = jnp.transpose(x[:, :, :K1], (0, 2, 1))
    body = functools.partial(_gnn_body, n=N, alpha=alpha, beta=beta,
                             c_b=c_b, c_r=c_r)
    return pl.pallas_call(
        body,
        grid=(B // S,),
        in_specs=[
            pl.BlockSpec((S, K1, N), lambda i: (i, 0, 0)),
            pl.BlockSpec((K1, H), lambda i: (0, 0)),
            pl.BlockSpec((H,), lambda i: (0,)),
            pl.BlockSpec((H, O), lambda i: (0, 0)),
            pl.BlockSpec((O,), lambda i: (0,)),
        ],
        out_specs=pl.BlockSpec((B, O), lambda i: (0, 0)),
        out_shape=jax.ShapeDtypeStruct((B, O), x.dtype),
    )(x_t, W1h, b1, W2, b2)


# speedup vs baseline: 1.1785x; 1.1785x over previous
import functools
import math

import jax
import jax.numpy as jnp
from jax.experimental import pallas as pl


def _gnn_body(x_ref, w1_ref, b1_ref, w2_ref, b2_ref, o_ref, *, n, alpha, beta, c_b, c_r):
    s, f, _ = x_ref.shape
    h = w1_ref.shape[1]
    xs = x_ref[...]
    w1 = w1_ref[...]
    nc = 200
    xw0 = jax.lax.dot_general(
        xs[:, :, :nc], w1,
        (((1,), (0,)), ((), ())),
        preferred_element_type=jnp.float32,
    )
    xb = xw0[:, 0, :]
    s1 = jnp.dot(jnp.sum(xs, axis=2), w1,
                 preferred_element_type=jnp.float32) - xb
    b1 = b1_ref[...]
    c = b1[None, :] + (2.0 * alpha) * xb
    t0 = jnp.maximum(c[:, None, :] + xw0, 0.0)
    S1 = jnp.sum(t0, axis=1) - t0[:, 0, :]
    for k in range(nc, n, nc):
        xwk = jax.lax.dot_general(
            xs[:, :, k:k + nc], w1,
            (((1,), (0,)), ((), ())),
            preferred_element_type=jnp.float32,
        )
        S1 = S1 + jnp.sum(jnp.maximum(c[:, None, :] + xwk, 0.0), axis=1)
    hb = jnp.maximum(b1[None, :] + (2.0 * alpha) * s1 + (2.0 * beta) * xb, 0.0)
    v = c_b * hb + c_r * S1
    i = pl.program_id(0)
    o_ref[pl.ds(i * s, s), :] = b2_ref[...][None, :] + jnp.dot(
        v, w2_ref[...], preferred_element_type=jnp.float32)


def kernel(x, W1, b1, W2, b2):
    B, N, F = x.shape
    K1, H = W1.shape
    O = W2.shape[1]
    W1p = jnp.zeros((F, H), W1.dtype).at[:K1, :].set(0.5 * W1)

    alpha = 1.0 / math.sqrt(2.0 * N)
    beta = 1.0 / N
    c_b = ((N - 1) * alpha + beta) / N
    c_r = (alpha + 0.5) / N

    S = 50
    assert B % S == 0
    x_t = jnp.transpose(x, (0, 2, 1))
    body = functools.partial(_gnn_body, n=N, alpha=alpha, beta=beta,
                             c_b=c_b, c_r=c_r)
    return pl.pallas_call(
        body,
        grid=(B // S,),
        in_specs=[
            pl.BlockSpec((S, F, N), lambda i: (i, 0, 0)),
            pl.BlockSpec((F, H), lambda i: (0, 0)),
            pl.BlockSpec((H,), lambda i: (0,)),
            pl.BlockSpec((H, O), lambda i: (0, 0)),
            pl.BlockSpec((O,), lambda i: (0,)),
        ],
        out_specs=pl.BlockSpec((B, O), lambda i: (0, 0)),
        out_shape=jax.ShapeDtypeStruct((B, O), x.dtype),
    )(x_t, W1p, b1, W2, b2)
